# baseline (speedup 1.0000x reference)
import jax
import jax.numpy as jnp
from jax.experimental import pallas as pl
from jax.experimental.pallas import tpu as pltpu


def _update_body(x_hbm, out_hbm, sem):
    cp = pltpu.make_async_copy(x_hbm, out_hbm, sem)
    cp.start()
    cp.wait()


def kernel(x, _, edge_index, W1, b1, W2, b2):
    return pl.pallas_call(
        _update_body,
        in_specs=[pl.BlockSpec(memory_space=pl.ANY)],
        out_specs=pl.BlockSpec(memory_space=pl.ANY),
        out_shape=jax.ShapeDtypeStruct(x.shape, x.dtype),
        scratch_shapes=[pltpu.SemaphoreType.DMA],
    )(x)


# speedup vs baseline: 18.5415x; 18.5415x over previous
import jax
import jax.numpy as jnp
from jax.experimental import pallas as pl
from jax.experimental.pallas import tpu as pltpu

_BLOCK_ROWS = 1000


def _update_body(x_ref, out_ref):
    out_ref[...] = x_ref[...]


def kernel(x, _, edge_index, W1, b1, W2, b2):
    n, d = x.shape
    rows = _BLOCK_ROWS if n % _BLOCK_ROWS == 0 else n
    return pl.pallas_call(
        _update_body,
        grid=(n // rows,),
        in_specs=[pl.BlockSpec((rows, d), lambda i: (i, 0))],
        out_specs=pl.BlockSpec((rows, d), lambda i: (i, 0)),
        out_shape=jax.ShapeDtypeStruct(x.shape, x.dtype),
        compiler_params=pltpu.CompilerParams(
            dimension_semantics=("arbitrary",),
        ),
    )(x)


# speedup vs baseline: 36.8394x; 1.9869x over previous
import jax
import jax.numpy as jnp
from jax.experimental import pallas as pl
from jax.experimental.pallas import tpu as pltpu

_BLOCK_ROWS = 5000


def _update_body(x_ref, out_ref):
    out_ref[...] = x_ref[...]


def kernel(x, _, edge_index, W1, b1, W2, b2):
    n, d = x.shape
    rows = _BLOCK_ROWS if n % _BLOCK_ROWS == 0 else n
    return pl.pallas_call(
        _update_body,
        grid=(n // rows,),
        in_specs=[pl.BlockSpec((rows, d), lambda i: (i, 0))],
        out_specs=pl.BlockSpec((rows, d), lambda i: (i, 0)),
        out_shape=jax.ShapeDtypeStruct(x.shape, x.dtype),
        compiler_params=pltpu.CompilerParams(
            dimension_semantics=("arbitrary",),
        ),
    )(x)
